# baseline (speedup 1.0000x reference)
import functools

import jax
import jax.numpy as jnp
from jax import lax
from jax.experimental import pallas as pl
from jax.experimental.pallas import tpu as pltpu
from jax.experimental.pallas import tpu_sc as plsc

N_E = 8192
E_DIM = 256
BETA = 0.25

BM = 1024
BK = 512
NK = N_E // BK


def _dist_argmin_body(zfs_ref, cb_ref, idx_ref, loss_ref,
                      z2_ref, c2_ref, minv_ref, mini_ref, iota_ref):
    m = pl.program_id(0)
    k = pl.program_id(1)
    nm = pl.num_programs(0)

    @pl.when(jnp.logical_and(m == 0, k == 0))
    def _():
        iota_ref[...] = lax.broadcasted_iota(
            jnp.int32, (BM, 128), 1).astype(jnp.float32)

    @pl.when(k == 0)
    def _():
        zfs = zfs_ref[...]
        z2_ref[...] = 0.25 * jnp.sum(zfs * zfs, axis=1, keepdims=True)

    @pl.when(m == 0)
    def _():
        cb = cb_ref[...]
        c2_ref[k] = jnp.sum(cb * cb, axis=1)[None, :]

    mm = lax.dot_general(zfs_ref[...], cb_ref[...], (((1,), (1,)), ((), ())),
                         preferred_element_type=jnp.float32)
    t1 = z2_ref[...] + c2_ref[k]
    ncol = BK // 128
    rmin = t1[:, 0:128] + mm[:, 0:128]
    ridx = jnp.zeros((BM, 128), jnp.float32)
    for i in range(1, ncol):
        col = t1[:, i * 128:(i + 1) * 128] + mm[:, i * 128:(i + 1) * 128]
        cm = col < rmin
        rmin = jnp.where(cm, col, rmin)
        ridx = jnp.where(cm, float(i), ridx)
    jidx = ridx * 128.0 + iota_ref[...]
    lmin = jnp.min(rmin, axis=1, keepdims=True)
    cand = jnp.where(rmin == lmin, jidx, float(N_E))
    lidx = jnp.min(cand, axis=1, keepdims=True) + k.astype(jnp.float32) * BK

    @pl.when(k == 0)
    def _():
        minv_ref[...] = lmin
        mini_ref[...] = lidx

    @pl.when(k > 0)
    def _():
        upd = lmin < minv_ref[...]
        minv_ref[...] = jnp.where(upd, lmin, minv_ref[...])
        mini_ref[...] = jnp.where(upd, lidx, mini_ref[...])

    @pl.when(k == NK - 1)
    def _():
        idx_ref[...] = mini_ref[...].astype(jnp.int32)
        s = jnp.sum(minv_ref[...])

        @pl.when(m == 0)
        def _():
            loss_ref[0, 0] = s

        @pl.when(m > 0)
        def _():
            loss_ref[0, 0] = loss_ref[0, 0] + s

        @pl.when(m == nm - 1)
        def _():
            scale = (1.0 + BETA) / float(N_E * E_DIM)
            loss_ref[0, 0] = loss_ref[0, 0] * scale


def _dist_argmin(zfs, codebook):
    n_tok = zfs.shape[0]
    grid = (n_tok // BM, NK)
    return pl.pallas_call(
        _dist_argmin_body,
        grid=grid,
        in_specs=[
            pl.BlockSpec((BM, E_DIM), lambda m, k: (m, 0)),
            pl.BlockSpec((BK, E_DIM), lambda m, k: (k, 0)),
        ],
        out_specs=[
            pl.BlockSpec((BM, 1), lambda m, k: (m, 0)),
            pl.BlockSpec(memory_space=pltpu.SMEM),
        ],
        out_shape=[
            jax.ShapeDtypeStruct((n_tok, 1), jnp.int32),
            jax.ShapeDtypeStruct((1, 1), jnp.float32),
        ],
        scratch_shapes=[
            pltpu.VMEM((BM, 1), jnp.float32),
            pltpu.VMEM((NK, 1, BK), jnp.float32),
            pltpu.VMEM((BM, 1), jnp.float32),
            pltpu.VMEM((BM, 1), jnp.float32),
            pltpu.VMEM((BM, 128), jnp.float32),
        ],
        compiler_params=pltpu.CompilerParams(
            dimension_semantics=("arbitrary", "arbitrary"),
        ),
    )(zfs, codebook)


_NC = 2
_NS = 16
_NW = _NC * _NS
_TOK = 8192
_BPW = _TOK // _NW
_GCH = 128


def _gather_body(cb_hbm, idx_hbm, out_hbm, idx_v, rows_v, sem):
    wid = lax.axis_index("s") * _NC + lax.axis_index("c")
    base = wid * _BPW
    for j in range(_BPW // _GCH):
        pltpu.sync_copy(idx_hbm.at[pl.ds(base + j * _GCH, _GCH)], idx_v.at[j])
        pltpu.async_copy(cb_hbm.at[idx_v.at[j]],
                         rows_v.at[pl.ds(j * _GCH, _GCH)], sem).wait()
    pltpu.sync_copy(rows_v, out_hbm.at[pl.ds(base, _BPW)])


@functools.cache
def _sc_gather_fn():
    return pl.kernel(
        _gather_body,
        out_type=jax.ShapeDtypeStruct((_TOK, E_DIM), jnp.float32),
        mesh=plsc.VectorSubcoreMesh(core_axis_name="c", subcore_axis_name="s"),
        scratch_types=[
            pltpu.VMEM((_BPW // _GCH, _GCH), jnp.int32),
            pltpu.VMEM((_BPW, E_DIM), jnp.float32),
            pltpu.SemaphoreType.DMA,
        ],
    )


def _st_body(z_ref, zq_ref, out_ref):
    z = z_ref[0]
    zqt = zq_ref[...].T
    out_ref[0] = z + (zqt - z)


def _st_transpose(z3, zq):
    b, c, hw = z3.shape
    return pl.pallas_call(
        _st_body,
        grid=(b,),
        in_specs=[
            pl.BlockSpec((1, c, hw), lambda i: (i, 0, 0)),
            pl.BlockSpec((hw, c), lambda i: (i, 0)),
        ],
        out_specs=pl.BlockSpec((1, c, hw), lambda i: (i, 0, 0)),
        out_shape=jax.ShapeDtypeStruct((b, c, hw), jnp.float32),
    )(z3, zq)


def kernel(z, codebook):
    b, c, h, w = z.shape
    zfs = jnp.transpose(z, (0, 2, 3, 1)).reshape(-1, E_DIM) * -2.0
    idx2, loss = _dist_argmin(zfs, codebook)
    idx = idx2.reshape(-1)
    zq = _sc_gather_fn()(codebook, idx)
    z3 = z.reshape(b, c, h * w)
    out3 = _st_transpose(z3, zq)
    z_q_out = out3.reshape(b, c, h, w)
    return (z_q_out, loss[0, 0], idx)


# speedup vs baseline: 1.1461x; 1.1461x over previous
import functools

import jax
import jax.numpy as jnp
from jax import lax
from jax.experimental import pallas as pl
from jax.experimental.pallas import tpu as pltpu
from jax.experimental.pallas import tpu_sc as plsc

N_E = 8192
E_DIM = 256
BETA = 0.25

BM = 1024
BK = 512
NK = N_E // BK


def _dist_argmin_body(zfs_ref, cb_ref, idx_ref, loss_ref,
                      z2_ref, c2_ref, minv_ref, mini_ref, iota_ref):
    m = pl.program_id(0)
    k = pl.program_id(1)
    nm = pl.num_programs(0)

    @pl.when(jnp.logical_and(m == 0, k == 0))
    def _():
        iota_ref[...] = lax.broadcasted_iota(
            jnp.int32, (BM, 128), 1).astype(jnp.float32)

    @pl.when(k == 0)
    def _():
        zfs = zfs_ref[...]
        z2 = 0.25 * jnp.sum(zfs * zfs, axis=1, keepdims=True)
        z2_ref[...] = jnp.broadcast_to(z2, (BM, 128))

    @pl.when(m == 0)
    def _():
        cb = cb_ref[...]
        c2 = jnp.sum(cb * cb, axis=1)[None, :]
        c2_ref[k] = jnp.broadcast_to(c2, (8, BK))

    @pl.when(k == 0)
    def _():
        minv_ref[...] = jnp.full((BM, 128), jnp.inf, jnp.float32)
        mini_ref[...] = jnp.zeros((BM, 128), jnp.float32)

    mm = lax.dot_general(zfs_ref[...], cb_ref[...], (((1,), (1,)), ((), ())),
                         preferred_element_type=jnp.float32)
    z2b = z2_ref[...]
    c2b = c2_ref[k]
    ncol = BK // 128

    rmin = minv_ref[...]
    ridx = mini_ref[...]
    for i in range(ncol):
        c2c = jnp.broadcast_to(c2b[:, i * 128:(i + 1) * 128]
                               .reshape(1, 8, 128), (BM // 8, 8, 128))
        t1 = z2b + c2c.reshape(BM, 128)
        col = t1 + mm[:, i * 128:(i + 1) * 128]
        cm = col < rmin
        rmin = jnp.where(cm, col, rmin)
        ridx = jnp.where(cm, (k * ncol + i).astype(jnp.float32), ridx)
    minv_ref[...] = rmin
    mini_ref[...] = ridx

    @pl.when(k == NK - 1)
    def _():
        lmin = jnp.min(rmin, axis=1, keepdims=True)
        jidx = ridx * 128.0 + iota_ref[...]
        cand = jnp.where(rmin == lmin, jidx, float(N_E))
        lidx = jnp.min(cand, axis=1, keepdims=True)
        idx_ref[...] = lidx.astype(jnp.int32)
        s = jnp.sum(lmin)

        @pl.when(m == 0)
        def _():
            loss_ref[0, 0] = s

        @pl.when(m > 0)
        def _():
            loss_ref[0, 0] = loss_ref[0, 0] + s

        @pl.when(m == nm - 1)
        def _():
            scale = (1.0 + BETA) / float(N_E * E_DIM)
            loss_ref[0, 0] = loss_ref[0, 0] * scale


def _dist_argmin(zfs, codebook):
    n_tok = zfs.shape[0]
    grid = (n_tok // BM, NK)
    return pl.pallas_call(
        _dist_argmin_body,
        grid=grid,
        in_specs=[
            pl.BlockSpec((BM, E_DIM), lambda m, k: (m, 0)),
            pl.BlockSpec((BK, E_DIM), lambda m, k: (k, 0)),
        ],
        out_specs=[
            pl.BlockSpec((BM, 1), lambda m, k: (m, 0)),
            pl.BlockSpec(memory_space=pltpu.SMEM),
        ],
        out_shape=[
            jax.ShapeDtypeStruct((n_tok, 1), jnp.int32),
            jax.ShapeDtypeStruct((1, 1), jnp.float32),
        ],
        scratch_shapes=[
            pltpu.VMEM((BM, 128), jnp.float32),
            pltpu.VMEM((NK, 8, BK), jnp.float32),
            pltpu.VMEM((BM, 128), jnp.float32),
            pltpu.VMEM((BM, 128), jnp.float32),
            pltpu.VMEM((BM, 128), jnp.float32),
        ],
        compiler_params=pltpu.CompilerParams(
            dimension_semantics=("arbitrary", "arbitrary"),
        ),
    )(zfs, codebook)


_NC = 2
_NS = 16
_NW = _NC * _NS
_TOK = 8192
_BPW = _TOK // _NW
_GCH = 128


def _gather_body(cb_hbm, idx_hbm, out_hbm, idx_v, rows_v, sem):
    wid = lax.axis_index("s") * _NC + lax.axis_index("c")
    base = wid * _BPW
    for j in range(_BPW // _GCH):
        pltpu.sync_copy(idx_hbm.at[pl.ds(base + j * _GCH, _GCH)], idx_v.at[j])
        pltpu.async_copy(cb_hbm.at[idx_v.at[j]],
                         rows_v.at[pl.ds(j * _GCH, _GCH)], sem).wait()
    pltpu.sync_copy(rows_v, out_hbm.at[pl.ds(base, _BPW)])


@functools.cache
def _sc_gather_fn():
    return pl.kernel(
        _gather_body,
        out_type=jax.ShapeDtypeStruct((_TOK, E_DIM), jnp.float32),
        mesh=plsc.VectorSubcoreMesh(core_axis_name="c", subcore_axis_name="s"),
        scratch_types=[
            pltpu.VMEM((_BPW // _GCH, _GCH), jnp.int32),
            pltpu.VMEM((_BPW, E_DIM), jnp.float32),
            pltpu.SemaphoreType.DMA,
        ],
    )


def _st_body(z_ref, zq_ref, out_ref):
    z = z_ref[0]
    zqt = zq_ref[...].T
    out_ref[0] = z + (zqt - z)


def _st_transpose(z3, zq):
    b, c, hw = z3.shape
    return pl.pallas_call(
        _st_body,
        grid=(b,),
        in_specs=[
            pl.BlockSpec((1, c, hw), lambda i: (i, 0, 0)),
            pl.BlockSpec((hw, c), lambda i: (i, 0)),
        ],
        out_specs=pl.BlockSpec((1, c, hw), lambda i: (i, 0, 0)),
        out_shape=jax.ShapeDtypeStruct((b, c, hw), jnp.float32),
    )(z3, zq)


def kernel(z, codebook):
    b, c, h, w = z.shape
    zfs = jnp.transpose(z, (0, 2, 3, 1)).reshape(-1, E_DIM) * -2.0
    idx2, loss = _dist_argmin(zfs, codebook)
    idx = idx2.reshape(-1)
    zq = _sc_gather_fn()(codebook, idx)
    z3 = z.reshape(b, c, h * w)
    out3 = _st_transpose(z3, zq)
    z_q_out = out3.reshape(b, c, h, w)
    return (z_q_out, loss[0, 0], idx)


# speedup vs baseline: 1.5509x; 1.3532x over previous
import functools

import jax
import jax.numpy as jnp
from jax import lax
from jax.experimental import pallas as pl
from jax.experimental.pallas import tpu as pltpu
from jax.experimental.pallas import tpu_sc as plsc

N_E = 8192
E_DIM = 256
BETA = 0.25

BM = 1024
BKB = 2048
SUB = 512
NKB = N_E // BKB


def _dist_argmin_body(zfs_ref, cb_ref, idx_ref, loss_ref,
                      z2_ref, c2_ref, minv_ref, mini_ref, iota_ref):
    m = pl.program_id(0)
    k = pl.program_id(1)
    nm = pl.num_programs(0)

    @pl.when(jnp.logical_and(m == 0, k == 0))
    def _():
        iota_ref[...] = lax.broadcasted_iota(
            jnp.int32, (BM, 128), 1).astype(jnp.float32)

    @pl.when(k == 0)
    def _():
        zfs = zfs_ref[...]
        z2 = 0.25 * jnp.sum(zfs * zfs, axis=1, keepdims=True)
        z2_ref[...] = jnp.broadcast_to(z2, (BM, 128))
        minv_ref[...] = jnp.full((BM, 128), jnp.inf, jnp.float32)
        mini_ref[...] = jnp.zeros((BM, 128), jnp.float32)

    @pl.when(m == 0)
    def _():
        cb = cb_ref[...]
        c2 = jnp.sum(cb * cb, axis=1)[None, :]
        c2_ref[k] = jnp.broadcast_to(c2, (8, BKB))

    z2b = z2_ref[...]
    c2b = c2_ref[k]

    rmin = minv_ref[...]
    ridx = mini_ref[...]
    for s in range(BKB // SUB):
        mm = lax.dot_general(
            zfs_ref[...], cb_ref[s * SUB:(s + 1) * SUB, :],
            (((1,), (1,)), ((), ())),
            preferred_element_type=jnp.float32)
        for i in range(SUB // 128):
            ci = s * (SUB // 128) + i
            c2c = jnp.broadcast_to(c2b[:, ci * 128:(ci + 1) * 128]
                                   .reshape(1, 8, 128), (BM // 8, 8, 128))
            t1 = z2b + c2c.reshape(BM, 128)
            col = t1 + mm[:, i * 128:(i + 1) * 128]
            cm = col < rmin
            rmin = jnp.where(cm, col, rmin)
            ridx = jnp.where(
                cm, (k * (BKB // 128) + ci).astype(jnp.float32), ridx)
    minv_ref[...] = rmin
    mini_ref[...] = ridx

    @pl.when(k == NKB - 1)
    def _():
        lmin = jnp.min(rmin, axis=1, keepdims=True)
        jidx = ridx * 128.0 + iota_ref[...]
        cand = jnp.where(rmin == lmin, jidx, float(N_E))
        lidx = jnp.min(cand, axis=1, keepdims=True)
        idx_ref[...] = lidx.astype(jnp.int32)
        s = jnp.sum(lmin)

        @pl.when(m == 0)
        def _():
            loss_ref[0, 0] = s

        @pl.when(m > 0)
        def _():
            loss_ref[0, 0] = loss_ref[0, 0] + s

        @pl.when(m == nm - 1)
        def _():
            scale = (1.0 + BETA) / float(N_E * E_DIM)
            loss_ref[0, 0] = loss_ref[0, 0] * scale


def _dist_argmin(zfs, codebook):
    n_tok = zfs.shape[0]
    grid = (n_tok // BM, NKB)
    return pl.pallas_call(
        _dist_argmin_body,
        grid=grid,
        in_specs=[
            pl.BlockSpec((BM, E_DIM), lambda m, k: (m, 0)),
            pl.BlockSpec((BKB, E_DIM), lambda m, k: (k, 0)),
        ],
        out_specs=[
            pl.BlockSpec((BM, 1), lambda m, k: (m, 0)),
            pl.BlockSpec(memory_space=pltpu.SMEM),
        ],
        out_shape=[
            jax.ShapeDtypeStruct((n_tok, 1), jnp.int32),
            jax.ShapeDtypeStruct((1, 1), jnp.float32),
        ],
        scratch_shapes=[
            pltpu.VMEM((BM, 128), jnp.float32),
            pltpu.VMEM((NKB, 8, BKB), jnp.float32),
            pltpu.VMEM((BM, 128), jnp.float32),
            pltpu.VMEM((BM, 128), jnp.float32),
            pltpu.VMEM((BM, 128), jnp.float32),
        ],
        compiler_params=pltpu.CompilerParams(
            dimension_semantics=("arbitrary", "arbitrary"),
        ),
    )(zfs, codebook)


_NC = 2
_NS = 16
_NW = _NC * _NS
_TOK = 8192
_BPW = _TOK // _NW
_GCH = 128


def _gather_body(cb_hbm, idx_hbm, out_hbm, idx_v, rows_v, sem):
    wid = lax.axis_index("s") * _NC + lax.axis_index("c")
    base = wid * _BPW
    for j in range(_BPW // _GCH):
        pltpu.sync_copy(idx_hbm.at[pl.ds(base + j * _GCH, _GCH)], idx_v.at[j])
        pltpu.async_copy(cb_hbm.at[idx_v.at[j]],
                         rows_v.at[pl.ds(j * _GCH, _GCH)], sem).wait()
    pltpu.sync_copy(rows_v, out_hbm.at[pl.ds(base, _BPW)])


@functools.cache
def _sc_gather_fn():
    return pl.kernel(
        _gather_body,
        out_type=jax.ShapeDtypeStruct((_TOK, E_DIM), jnp.float32),
        mesh=plsc.VectorSubcoreMesh(core_axis_name="c", subcore_axis_name="s"),
        scratch_types=[
            pltpu.VMEM((_BPW // _GCH, _GCH), jnp.int32),
            pltpu.VMEM((_BPW, E_DIM), jnp.float32),
            pltpu.SemaphoreType.DMA,
        ],
    )


def _st_body(z_ref, zq_ref, out_ref):
    z = z_ref[0]
    zqt = zq_ref[...].T
    out_ref[0] = z + (zqt - z)


def _st_transpose(z3, zq):
    b, c, hw = z3.shape
    return pl.pallas_call(
        _st_body,
        grid=(b,),
        in_specs=[
            pl.BlockSpec((1, c, hw), lambda i: (i, 0, 0)),
            pl.BlockSpec((hw, c), lambda i: (i, 0)),
        ],
        out_specs=pl.BlockSpec((1, c, hw), lambda i: (i, 0, 0)),
        out_shape=jax.ShapeDtypeStruct((b, c, hw), jnp.float32),
    )(z3, zq)


def kernel(z, codebook):
    b, c, h, w = z.shape
    zfs = jnp.transpose(z, (0, 2, 3, 1)).reshape(-1, E_DIM) * -2.0
    idx2, loss = _dist_argmin(zfs, codebook)
    idx = idx2.reshape(-1)
    zq = _sc_gather_fn()(codebook, idx)
    z3 = z.reshape(b, c, h * w)
    out3 = _st_transpose(z3, zq)
    z_q_out = out3.reshape(b, c, h, w)
    return (z_q_out, loss[0, 0], idx)


# speedup vs baseline: 1.5965x; 1.0294x over previous
import functools

import jax
import jax.numpy as jnp
from jax import lax
from jax.experimental import pallas as pl
from jax.experimental.pallas import tpu as pltpu
from jax.experimental.pallas import tpu_sc as plsc

N_E = 8192
E_DIM = 256
BETA = 0.25

BM = 1024
BKB = 2048
SUB = 512
NKB = N_E // BKB


def _dist_argmin_body(zfs_ref, cb_ref, idx_ref, loss_ref,
                      z2_ref, c2_ref, iota_ref):
    m = pl.program_id(0)
    nm = pl.num_programs(0)

    @pl.when(m == 0)
    def _():
        iota_ref[...] = lax.broadcasted_iota(
            jnp.int32, (BM, 128), 1).astype(jnp.float32)
        cb = cb_ref[...]
        c2 = jnp.sum(cb * cb, axis=1)[None, :]
        c2_ref[...] = jnp.broadcast_to(c2, (8, N_E))

    zfs = zfs_ref[...]
    z2 = 0.25 * jnp.sum(zfs * zfs, axis=1, keepdims=True)
    z2_ref[...] = jnp.broadcast_to(z2, (BM, 128))

    z2b = z2_ref[...]
    rmin = jnp.full((BM, 128), jnp.inf, jnp.float32)
    ridx = jnp.zeros((BM, 128), jnp.float32)
    for s in range(N_E // SUB):
        mm = lax.dot_general(
            zfs, cb_ref[s * SUB:(s + 1) * SUB, :],
            (((1,), (1,)), ((), ())),
            preferred_element_type=jnp.float32)
        for i in range(SUB // 128):
            ci = s * (SUB // 128) + i
            c2c = jnp.broadcast_to(
                c2_ref[:, ci * 128:(ci + 1) * 128]
                .reshape(1, 8, 128), (BM // 8, 8, 128))
            t1 = z2b + c2c.reshape(BM, 128)
            col = t1 + mm[:, i * 128:(i + 1) * 128]
            cm = col < rmin
            rmin = jnp.where(cm, col, rmin)
            ridx = jnp.where(cm, float(ci), ridx)

    lmin = jnp.min(rmin, axis=1, keepdims=True)
    jidx = ridx * 128.0 + iota_ref[...]
    cand = jnp.where(rmin == lmin, jidx, float(N_E))
    lidx = jnp.min(cand, axis=1, keepdims=True)
    idx_ref[...] = lidx.astype(jnp.int32)
    acc = jnp.sum(lmin)

    @pl.when(m == 0)
    def _():
        loss_ref[0, 0] = acc

    @pl.when(m > 0)
    def _():
        loss_ref[0, 0] = loss_ref[0, 0] + acc

    @pl.when(m == nm - 1)
    def _():
        scale = (1.0 + BETA) / float(N_E * E_DIM)
        loss_ref[0, 0] = loss_ref[0, 0] * scale


def _dist_argmin(zfs, codebook):
    n_tok = zfs.shape[0]
    grid = (n_tok // BM,)
    return pl.pallas_call(
        _dist_argmin_body,
        grid=grid,
        in_specs=[
            pl.BlockSpec((BM, E_DIM), lambda m: (m, 0)),
            pl.BlockSpec((N_E, E_DIM), lambda m: (0, 0)),
        ],
        out_specs=[
            pl.BlockSpec((BM, 1), lambda m: (m, 0)),
            pl.BlockSpec(memory_space=pltpu.SMEM),
        ],
        out_shape=[
            jax.ShapeDtypeStruct((n_tok, 1), jnp.int32),
            jax.ShapeDtypeStruct((1, 1), jnp.float32),
        ],
        scratch_shapes=[
            pltpu.VMEM((BM, 128), jnp.float32),
            pltpu.VMEM((8, N_E), jnp.float32),
            pltpu.VMEM((BM, 128), jnp.float32),
        ],
        compiler_params=pltpu.CompilerParams(
            dimension_semantics=("arbitrary",),
        ),
    )(zfs, codebook)


_NC = 2
_NS = 16
_NW = _NC * _NS
_TOK = 8192
_BPW = _TOK // _NW
_GCH = 128


def _gather_body(cb_hbm, idx_hbm, out_hbm, idx_v, rows_v, sem):
    wid = lax.axis_index("s") * _NC + lax.axis_index("c")
    base = wid * _BPW
    for j in range(_BPW // _GCH):
        pltpu.sync_copy(idx_hbm.at[pl.ds(base + j * _GCH, _GCH)], idx_v.at[j])
        pltpu.async_copy(cb_hbm.at[idx_v.at[j]],
                         rows_v.at[pl.ds(j * _GCH, _GCH)], sem).wait()
    pltpu.sync_copy(rows_v, out_hbm.at[pl.ds(base, _BPW)])


@functools.cache
def _sc_gather_fn():
    return pl.kernel(
        _gather_body,
        out_type=jax.ShapeDtypeStruct((_TOK, E_DIM), jnp.float32),
        mesh=plsc.VectorSubcoreMesh(core_axis_name="c", subcore_axis_name="s"),
        scratch_types=[
            pltpu.VMEM((_BPW // _GCH, _GCH), jnp.int32),
            pltpu.VMEM((_BPW, E_DIM), jnp.float32),
            pltpu.SemaphoreType.DMA,
        ],
    )


def _st_body(z_ref, zq_ref, out_ref):
    z = z_ref[0]
    zqt = zq_ref[...].T
    out_ref[0] = z + (zqt - z)


def _st_transpose(z3, zq):
    b, c, hw = z3.shape
    return pl.pallas_call(
        _st_body,
        grid=(b,),
        in_specs=[
            pl.BlockSpec((1, c, hw), lambda i: (i, 0, 0)),
            pl.BlockSpec((hw, c), lambda i: (i, 0)),
        ],
        out_specs=pl.BlockSpec((1, c, hw), lambda i: (i, 0, 0)),
        out_shape=jax.ShapeDtypeStruct((b, c, hw), jnp.float32),
    )(z3, zq)


def kernel(z, codebook):
    b, c, h, w = z.shape
    zfs = jnp.transpose(z, (0, 2, 3, 1)).reshape(-1, E_DIM) * -2.0
    idx2, loss = _dist_argmin(zfs, codebook)
    idx = idx2.reshape(-1)
    zq = _sc_gather_fn()(codebook, idx)
    z3 = z.reshape(b, c, h * w)
    out3 = _st_transpose(z3, zq)
    z_q_out = out3.reshape(b, c, h, w)
    return (z_q_out, loss[0, 0], idx)
